# baseline (speedup 1.0000x reference)
import functools
import math

import jax
import jax.numpy as jnp
from jax import lax
from jax.experimental import pallas as pl
from jax.experimental.pallas import tpu as pltpu
from jax.experimental.pallas import tpu_sc as plsc

NB = 500
NBP = 512
NSAMP = 131072
NTILES = 16
CHUNK = NSAMP // NTILES
VECS = CHUNK // 16
TAPS = 3
PREF = 2.0 / math.sqrt(2.0 * math.pi)
HALF_LOG_2PI = 0.5 * math.log(2.0 * math.pi)


def _sc_hist_body(x2_hbm, counts_out, mm_out,
                  chunk_v, hist_v, counts_v, ctab_v, vec16_v, mmall_v, red_v,
                  out32_v, sh_mm, sh_counts):
    c = lax.axis_index("c")
    s = lax.axis_index("s")
    lane = lax.iota(jnp.int32, 16)

    pltpu.sync_copy(x2_hbm.at[pl.ds(c * NSAMP + s * CHUNK, CHUNK)], chunk_v)

    def mm_step(i, carry):
        mn, mx = carry
        for k in range(4):
            v = chunk_v[pl.ds(i * 64 + k * 16, 16)]
            mn = jnp.minimum(mn, v)
            mx = jnp.maximum(mx, v)
        return mn, mx

    v0 = chunk_v[pl.ds(0, 16)]
    mn_v, mx_v = lax.fori_loop(0, VECS // 4, mm_step, (v0, v0))
    vec16_v[...] = mn_v
    pltpu.sync_copy(vec16_v, sh_mm.at[pl.ds(s * 32, 16)])
    vec16_v[...] = mx_v
    pltpu.sync_copy(vec16_v, sh_mm.at[pl.ds(s * 32 + 16, 16)])
    plsc.subcore_barrier()
    pltpu.sync_copy(sh_mm, mmall_v)
    mn16 = mmall_v[pl.ds(0, 16)]
    mx16 = mmall_v[pl.ds(16, 16)]
    for r in range(1, NTILES):
        mn16 = jnp.minimum(mn16, mmall_v[pl.ds(r * 32, 16)])
        mx16 = jnp.maximum(mx16, mmall_v[pl.ds(r * 32 + 16, 16)])

    def lanefold(v, op):
        for shift in (8, 4, 2, 1):
            vec16_v[...] = v
            idx = (lane + shift) & 15
            v = op(v, plsc.load_gather(vec16_v, [idx]))
        return v

    gmn_v = lanefold(mn16, jnp.minimum)
    gmx_v = lanefold(mx16, jnp.maximum)
    dr_v = gmx_v - gmn_v
    bw_v = dr_v / jnp.float32(500.0)
    inv_bw_v = jnp.float32(500.0) / dr_v
    std_v = bw_v * jnp.float32(0.5)
    d2_v = (std_v * std_v) * jnp.float32(2.0)
    r_hi = jnp.float32(1.0) / d2_v
    p = r_hi * d2_v
    ah = r_hi * jnp.float32(4097.0)
    ah = ah - (ah - r_hi)
    al = r_hi - ah
    bh = d2_v * jnp.float32(4097.0)
    bh = bh - (bh - d2_v)
    bl = d2_v - bh
    perr = ((ah * bh - p) + ah * bl + al * bh) + al * bl
    r_lo = ((jnp.float32(1.0) - p) - perr) * r_hi
    nr_hi = -r_hi
    nr_lo = -r_lo

    def ctab_step(k, carry):
        jf = (lane + k * 16).astype(jnp.float32)
        ctab_v[pl.ds(k * 16, 16)] = gmn_v + bw_v * (jf + jnp.float32(0.5))
        return carry

    lax.fori_loop(0, NBP // 16, ctab_step, 0)

    zero16 = jnp.zeros((16,), jnp.float32)

    def zero_step(i, carry):
        hist_v[pl.ds(i * 16, 16)] = zero16
        return carry

    lax.fori_loop(0, (16 * NBP) // 16, zero_step, 0)

    lane_base = lane * NBP

    def hist_step(i, carry):
        for k in range(2):
            x = chunk_v[pl.ds(i * 32 + k * 16, 16)]
            u = (x - gmn_v) * inv_bw_v
            base = u.astype(jnp.int32)
            for d in range(-TAPS, TAPS + 1):
                ji = base + d
                jc0 = jnp.clip(ji, 0, NBP - 1)
                cj = plsc.load_gather(ctab_v, [jc0])
                diff = x - cj
                sq = diff * diff
                q = sq * nr_hi + sq * nr_lo
                val = jnp.exp(q)
                m = (ji >= 0) & (ji < NB)
                plsc.addupdate_scatter(hist_v, [jc0 + lane_base], val, mask=m)
        return carry

    lax.fori_loop(0, VECS // 2, hist_step, 0)

    def lred_step(h, carry):
        a = hist_v[pl.ds(h * 16, 16)]
        for r in range(1, 16):
            a = a + hist_v[pl.ds(r * NBP + h * 16, 16)]
        counts_v[pl.ds(h * 16, 16)] = a
        return carry

    lax.fori_loop(0, NBP // 16, lred_step, 0)

    pltpu.sync_copy(counts_v, sh_counts.at[pl.ds(s * NBP, NBP)])
    plsc.subcore_barrier()
    for r in range(NTILES):
        pltpu.sync_copy(sh_counts.at[pl.ds(r * NBP + s * 32, 32)],
                        red_v.at[pl.ds(r * 32, 32)])
    for h in range(2):
        a = red_v[pl.ds(h * 16, 16)]
        for r in range(1, NTILES):
            a = a + red_v[pl.ds(r * 32 + h * 16, 16)]
        out32_v[pl.ds(h * 16, 16)] = a
    pltpu.sync_copy(out32_v, counts_out.at[pl.ds(c * NBP + s * 32, 32)])

    @pl.when(s == 0)
    def _():
        vec16_v[...] = jnp.where(lane == 0, gmn_v,
                                 jnp.where(lane == 1, gmx_v, jnp.float32(0.0)))
        pltpu.sync_copy(vec16_v, mm_out.at[pl.ds(c * 16, 16)])


_sc_hist = pl.kernel(
    _sc_hist_body,
    out_type=[
        jax.ShapeDtypeStruct((2 * NBP,), jnp.float32),
        jax.ShapeDtypeStruct((32,), jnp.float32),
    ],
    mesh=plsc.VectorSubcoreMesh(core_axis_name="c", subcore_axis_name="s"),
    compiler_params=pltpu.CompilerParams(needs_layout_passes=False),
    scratch_types=[
        pltpu.VMEM((CHUNK,), jnp.float32),
        pltpu.VMEM((16 * NBP,), jnp.float32),
        pltpu.VMEM((NBP,), jnp.float32),
        pltpu.VMEM((NBP,), jnp.float32),
        pltpu.VMEM((16,), jnp.float32),
        pltpu.VMEM((NTILES * 32,), jnp.float32),
        pltpu.VMEM((NTILES * 32,), jnp.float32),
        pltpu.VMEM((32,), jnp.float32),
        pltpu.VMEM_SHARED((NTILES * 32,), jnp.float32),
        pltpu.VMEM_SHARED((NTILES * NBP,), jnp.float32),
    ],
)


def _ts(a, b):
    s = a + b
    bb = s - a
    err = (a - (s - bb)) + (b - bb)
    return s, err


def _dadd(ah, al, bh, bl):
    s, e = _ts(ah, bh)
    return s, (al + bl) + e


def _split(a):
    c = a * jnp.float32(4097.0)
    t = c - a
    hi = c - t
    return hi, a - hi


def _two_prod(a, b):
    p = a * b
    ah, al = _split(a)
    bh, bl = _split(b)
    err = ((ah * bh - p) + ah * bl + al * bh) + al * bl
    return p, err


def _fold_rows(hi, lo):
    n = hi.shape[0]
    while n > 8:
        h = n // 2
        s, e = _ts(hi[:h], hi[h:n])
        lo = (lo[:h] + lo[h:n]) + e
        hi, n = s, h
    return hi, lo


def _fold_lanes(hi, lo):
    n = hi.shape[1]
    while n > 128:
        h = n // 2
        s, e = _ts(hi[:, :h], hi[:, h:n])
        lo = (lo[:, :h] + lo[:, h:n]) + e
        hi, n = s, h
    for sh in (64, 32, 16, 8, 4, 2, 1):
        rh = pltpu.roll(hi, sh, axis=1)
        rl = pltpu.roll(lo, sh, axis=1)
        s, e = _ts(hi, rh)
        lo = (lo + rl) + e
        hi = s
    return hi, lo


def _tail_body(counts_ref, mm_ref, cov_ref, out_ref, sc_hi, sc_lo):
    mnA = mm_ref[0, 0]
    mxA = mm_ref[0, 1]
    mnB = mm_ref[0, 16]
    mxB = mm_ref[0, 17]
    cov = cov_ref[0, 0]
    bwA = (mxA - mnA) / jnp.float32(500.0)
    bwB = (mxB - mnB) / jnp.float32(500.0)

    def centres(idx_f, idx_i):
        cA = mnA + bwA * (idx_f + jnp.float32(0.5))
        cB = mnB + bwB * ((idx_f - jnp.float32(512.0)) + jnp.float32(0.5))
        return jnp.where(idx_i < 512, cA, cB)

    def valid(idx_i):
        return (idx_i < NB) | ((idx_i >= 512) & (idx_i < 512 + NB))

    jcol = lax.broadcasted_iota(jnp.int32, (1, 2 * NBP), 1)
    jcol_f = jcol.astype(jnp.float32)
    tj = centres(jcol_f, jcol)
    colvalid = valid(jcol)

    irow = lax.broadcasted_iota(jnp.int32, (2 * NBP, 1), 0)
    irow_f = irow.astype(jnp.float32)
    tv = centres(irow_f, irow)
    rowvalid = valid(irow)

    diff = (tv - tj) / cov
    lp = jnp.float32(-0.5) * (diff * diff)
    lp = (lp - jnp.log(cov)) - jnp.float32(HALF_LOG_2PI)
    lp = jnp.where(rowvalid, lp, jnp.float32(0.0))

    def colsum(rows_hi):
        h8, l8 = _fold_rows(rows_hi, jnp.zeros_like(rows_hi))
        sc_hi[...] = h8
        sc_lo[...] = l8
        hi = sc_hi[0:1, :]
        lo = sc_lo[0:1, :]
        for r in range(1, 8):
            hi, e = _ts(hi, sc_hi[r:r + 1, :])
            lo = (lo + sc_lo[r:r + 1, :]) + e
        return hi, lo

    CAh, CAl = colsum(lp[:NBP])
    CBh, CBl = colsum(lp[NBP:])

    w = counts_ref[...] * (jnp.float32(PREF) / jnp.float32(NSAMP))
    w = jnp.where(colvalid, w, jnp.float32(0.0))
    wA = jnp.where(jcol < 512, w, jnp.float32(0.0))
    wB = w - wA

    def dot2(wv, ch, cl):
        return _fold_lanes(wv * ch, wv * cl)

    P1h, P1l = _dadd(*dot2(wA, CAh, CAl), *dot2(wB, CBh, CBl))
    P2h, P2l = _dadd(*dot2(wA, CBh, CBl), *dot2(wB, CAh, CAl))
    Sh, Sl = _fold_lanes(w, jnp.zeros_like(w))

    Sm1h = Sh - jnp.float32(1.0)
    t2h, t2e = _two_prod(Sm1h, P1h)
    t2l = (Sm1h * P1l + Sl * P1h) + t2e
    t3h, t3l = _dadd(P2h, P2l, -t2h, -t2l)
    res = (t3h + t3l) / (Sh + Sl)
    out_ref[...] = res[0:1, 0:1]


_tc_tail = pl.pallas_call(
    _tail_body,
    out_shape=jax.ShapeDtypeStruct((1, 1), jnp.float32),
    in_specs=[
        pl.BlockSpec(memory_space=pltpu.VMEM),
        pl.BlockSpec(memory_space=pltpu.SMEM),
        pl.BlockSpec(memory_space=pltpu.SMEM),
    ],
    out_specs=pl.BlockSpec(memory_space=pltpu.VMEM),
    scratch_shapes=[
        pltpu.VMEM((8, 2 * NBP), jnp.float32),
        pltpu.VMEM((8, 2 * NBP), jnp.float32),
    ],
)


def _weights_bins(counts_raw, lo, hi):
    bw = (hi - lo) / jnp.float32(500.0)
    std = jnp.float32(1.0) * bw / 2.0
    denom = jnp.float32(math.sqrt(2.0 * math.pi)) * std
    q_hi = bw / denom
    p = q_hi * denom
    dh, dl = _split(q_hi)
    eh, el = _split(denom)
    perr = ((dh * eh - p) + dh * el + dl * eh) + dl * el
    q_lo = ((bw - p) - perr) / denom
    w = (counts_raw * q_hi + counts_raw * q_lo) / jnp.float32(NSAMP)
    centres = lo + bw * (jnp.arange(NB, dtype=jnp.float32) + 0.5)
    return w, centres


def _log_likelihood_ref(bins, cov):
    val = bins[:, None]
    loc = bins[None, :]
    lp = -0.5 * jnp.square((val - loc) / cov) - jnp.log(cov) - jnp.float32(HALF_LOG_2PI)
    return lp.sum(axis=0)


def kernel(XA_1d, XB_1d, likelihood_cov):
    x2 = jnp.concatenate([jnp.squeeze(XA_1d, axis=1), jnp.squeeze(XB_1d, axis=1)])
    counts, mm = _sc_hist(x2)
    weights_A, bins_A = _weights_bins(counts[:NB], mm[0], mm[1])
    weights_B, bins_B = _weights_bins(counts[NBP:NBP + NB], mm[16], mm[17])
    cov = likelihood_cov
    weights_AB = jnp.concatenate([weights_A, weights_B]) / (weights_A.sum() + weights_B.sum())
    bins_AB = jnp.concatenate([bins_A, bins_B])
    avg_log_llhd_A = (_log_likelihood_ref(bins_A, cov) * weights_A).sum()
    avg_log_llhd_B = (_log_likelihood_ref(bins_B, cov) * weights_B).sum()
    avg_log_llhd_AB = (_log_likelihood_ref(bins_AB, cov) * weights_AB).sum()
    return avg_log_llhd_AB - avg_log_llhd_A - avg_log_llhd_B


# speedup vs baseline: 1.7788x; 1.7788x over previous
import functools
import math

import jax
import jax.numpy as jnp
from jax import lax
from jax.experimental import pallas as pl
from jax.experimental.pallas import tpu as pltpu
from jax.experimental.pallas import tpu_sc as plsc

NB = 500
NBP = 512
NSAMP = 131072
NTILES = 16
CHUNK = NSAMP // NTILES
VECS = CHUNK // 16
TAPS = 3
PREF = 2.0 / math.sqrt(2.0 * math.pi)
HALF_LOG_2PI = 0.5 * math.log(2.0 * math.pi)


def _sc_hist_body(x2_hbm, counts_out, mm_out,
                  chunk_v, hist_v, counts_v, vec16_v, mmall_v, red_v,
                  out32_v, sh_mm, sh_counts):
    c = lax.axis_index("c")
    s = lax.axis_index("s")
    lane = lax.iota(jnp.int32, 16)

    pltpu.sync_copy(x2_hbm.at[pl.ds(c * NSAMP + s * CHUNK, CHUNK)], chunk_v)

    def mm_step(i, carry):
        mn, mx = carry
        for k in range(4):
            v = chunk_v[pl.ds(i * 64 + k * 16, 16)]
            mn = jnp.minimum(mn, v)
            mx = jnp.maximum(mx, v)
        return mn, mx

    v0 = chunk_v[pl.ds(0, 16)]
    mn_v, mx_v = lax.fori_loop(0, VECS // 4, mm_step, (v0, v0))
    vec16_v[...] = mn_v
    pltpu.sync_copy(vec16_v, sh_mm.at[pl.ds(s * 32, 16)])
    vec16_v[...] = mx_v
    pltpu.sync_copy(vec16_v, sh_mm.at[pl.ds(s * 32 + 16, 16)])
    plsc.subcore_barrier()
    pltpu.sync_copy(sh_mm, mmall_v)
    mn16 = mmall_v[pl.ds(0, 16)]
    mx16 = mmall_v[pl.ds(16, 16)]
    for r in range(1, NTILES):
        mn16 = jnp.minimum(mn16, mmall_v[pl.ds(r * 32, 16)])
        mx16 = jnp.maximum(mx16, mmall_v[pl.ds(r * 32 + 16, 16)])

    def lanefold(v, op):
        for shift in (8, 4, 2, 1):
            vec16_v[...] = v
            idx = (lane + shift) & 15
            v = op(v, plsc.load_gather(vec16_v, [idx]))
        return v

    gmn_v = lanefold(mn16, jnp.minimum)
    gmx_v = lanefold(mx16, jnp.maximum)
    dr_v = gmx_v - gmn_v
    bw_v = dr_v / jnp.float32(500.0)
    inv_bw_v = jnp.float32(500.0) / dr_v
    std_v = bw_v * jnp.float32(0.5)
    d2_v = (std_v * std_v) * jnp.float32(2.0)
    r_hi = jnp.float32(1.0) / d2_v
    p = r_hi * d2_v
    ah = r_hi * jnp.float32(4097.0)
    ah = ah - (ah - r_hi)
    al = r_hi - ah
    bh = d2_v * jnp.float32(4097.0)
    bh = bh - (bh - d2_v)
    bl = d2_v - bh
    perr = ((ah * bh - p) + ah * bl + al * bh) + al * bl
    r_lo = ((jnp.float32(1.0) - p) - perr) * r_hi
    nr_hi = -r_hi
    nr_lo = -r_lo


    zero16 = jnp.zeros((16,), jnp.float32)

    def zero_step(i, carry):
        hist_v[pl.ds(i * 16, 16)] = zero16
        return carry

    lax.fori_loop(0, (16 * NBP) // 16, zero_step, 0)

    lane_base = lane * NBP

    def hist_step(i, carry):
        for k in range(2):
            x = chunk_v[pl.ds(i * 32 + k * 16, 16)]
            u = (x - gmn_v) * inv_bw_v
            base = u.astype(jnp.int32)
            for d in range(-TAPS, TAPS + 1):
                ji = base + d
                jc0 = jnp.clip(ji, 0, NBP - 1)
                cj = gmn_v + bw_v * (ji.astype(jnp.float32) + jnp.float32(0.5))
                diff = x - cj
                sq = diff * diff
                q = sq * nr_hi + sq * nr_lo
                val = jnp.exp(q)
                m = (ji >= 0) & (ji < NB)
                plsc.addupdate_scatter(hist_v, [jc0 + lane_base], val, mask=m)
        return carry

    lax.fori_loop(0, VECS // 2, hist_step, 0)

    def lred_step(h, carry):
        a = hist_v[pl.ds(h * 16, 16)]
        for r in range(1, 16):
            a = a + hist_v[pl.ds(r * NBP + h * 16, 16)]
        counts_v[pl.ds(h * 16, 16)] = a
        return carry

    lax.fori_loop(0, NBP // 16, lred_step, 0)

    pltpu.sync_copy(counts_v, sh_counts.at[pl.ds(s * NBP, NBP)])
    plsc.subcore_barrier()
    for r in range(NTILES):
        pltpu.sync_copy(sh_counts.at[pl.ds(r * NBP + s * 32, 32)],
                        red_v.at[pl.ds(r * 32, 32)])
    for h in range(2):
        a = red_v[pl.ds(h * 16, 16)]
        for r in range(1, NTILES):
            a = a + red_v[pl.ds(r * 32 + h * 16, 16)]
        out32_v[pl.ds(h * 16, 16)] = a
    pltpu.sync_copy(out32_v, counts_out.at[pl.ds(c * NBP + s * 32, 32)])

    @pl.when(s == 0)
    def _():
        vec16_v[...] = jnp.where(lane == 0, gmn_v,
                                 jnp.where(lane == 1, gmx_v, jnp.float32(0.0)))
        pltpu.sync_copy(vec16_v, mm_out.at[pl.ds(c * 16, 16)])


_sc_hist = pl.kernel(
    _sc_hist_body,
    out_type=[
        jax.ShapeDtypeStruct((2 * NBP,), jnp.float32),
        jax.ShapeDtypeStruct((32,), jnp.float32),
    ],
    mesh=plsc.VectorSubcoreMesh(core_axis_name="c", subcore_axis_name="s"),
    compiler_params=pltpu.CompilerParams(needs_layout_passes=False),
    scratch_types=[
        pltpu.VMEM((CHUNK,), jnp.float32),
        pltpu.VMEM((16 * NBP,), jnp.float32),
        pltpu.VMEM((NBP,), jnp.float32),
        pltpu.VMEM((16,), jnp.float32),
        pltpu.VMEM((NTILES * 32,), jnp.float32),
        pltpu.VMEM((NTILES * 32,), jnp.float32),
        pltpu.VMEM((32,), jnp.float32),
        pltpu.VMEM_SHARED((NTILES * 32,), jnp.float32),
        pltpu.VMEM_SHARED((NTILES * NBP,), jnp.float32),
    ],
)


def _ts(a, b):
    s = a + b
    bb = s - a
    err = (a - (s - bb)) + (b - bb)
    return s, err


def _dadd(ah, al, bh, bl):
    s, e = _ts(ah, bh)
    return s, (al + bl) + e


def _split(a):
    c = a * jnp.float32(4097.0)
    t = c - a
    hi = c - t
    return hi, a - hi


def _two_prod(a, b):
    p = a * b
    ah, al = _split(a)
    bh, bl = _split(b)
    err = ((ah * bh - p) + ah * bl + al * bh) + al * bl
    return p, err


def _fold_rows(hi, lo):
    n = hi.shape[0]
    while n > 8:
        h = n // 2
        s, e = _ts(hi[:h], hi[h:n])
        lo = (lo[:h] + lo[h:n]) + e
        hi, n = s, h
    return hi, lo


def _fold_lanes(hi, lo):
    n = hi.shape[1]
    while n > 128:
        h = n // 2
        s, e = _ts(hi[:, :h], hi[:, h:n])
        lo = (lo[:, :h] + lo[:, h:n]) + e
        hi, n = s, h
    for sh in (64, 32, 16, 8, 4, 2, 1):
        rh = pltpu.roll(hi, sh, axis=1)
        rl = pltpu.roll(lo, sh, axis=1)
        s, e = _ts(hi, rh)
        lo = (lo + rl) + e
        hi = s
    return hi, lo


def _tail_body(counts_ref, mm_ref, cov_ref, out_ref, sc_hi, sc_lo):
    mnA = mm_ref[0, 0]
    mxA = mm_ref[0, 1]
    mnB = mm_ref[0, 16]
    mxB = mm_ref[0, 17]
    cov = cov_ref[0, 0]
    bwA = (mxA - mnA) / jnp.float32(500.0)
    bwB = (mxB - mnB) / jnp.float32(500.0)

    def centres(idx_f, idx_i):
        cA = mnA + bwA * (idx_f + jnp.float32(0.5))
        cB = mnB + bwB * ((idx_f - jnp.float32(512.0)) + jnp.float32(0.5))
        return jnp.where(idx_i < 512, cA, cB)

    def valid(idx_i):
        return (idx_i < NB) | ((idx_i >= 512) & (idx_i < 512 + NB))

    jcol = lax.broadcasted_iota(jnp.int32, (1, 2 * NBP), 1)
    jcol_f = jcol.astype(jnp.float32)
    tj = centres(jcol_f, jcol)
    colvalid = valid(jcol)

    irow = lax.broadcasted_iota(jnp.int32, (2 * NBP, 1), 0)
    irow_f = irow.astype(jnp.float32)
    tv = centres(irow_f, irow)
    rowvalid = valid(irow)

    diff = (tv - tj) / cov
    lp = jnp.float32(-0.5) * (diff * diff)
    lp = (lp - jnp.log(cov)) - jnp.float32(HALF_LOG_2PI)
    lp = jnp.where(rowvalid, lp, jnp.float32(0.0))

    def colsum(rows_hi):
        h8, l8 = _fold_rows(rows_hi, jnp.zeros_like(rows_hi))
        sc_hi[...] = h8
        sc_lo[...] = l8
        hi = sc_hi[0:1, :]
        lo = sc_lo[0:1, :]
        for r in range(1, 8):
            hi, e = _ts(hi, sc_hi[r:r + 1, :])
            lo = (lo + sc_lo[r:r + 1, :]) + e
        return hi, lo

    CAh, CAl = colsum(lp[:NBP])
    CBh, CBl = colsum(lp[NBP:])

    w = counts_ref[...] * (jnp.float32(PREF) / jnp.float32(NSAMP))
    w = jnp.where(colvalid, w, jnp.float32(0.0))
    wA = jnp.where(jcol < 512, w, jnp.float32(0.0))
    wB = w - wA

    def dot2(wv, ch, cl):
        return _fold_lanes(wv * ch, wv * cl)

    P1h, P1l = _dadd(*dot2(wA, CAh, CAl), *dot2(wB, CBh, CBl))
    P2h, P2l = _dadd(*dot2(wA, CBh, CBl), *dot2(wB, CAh, CAl))
    Sh, Sl = _fold_lanes(w, jnp.zeros_like(w))

    Sm1h = Sh - jnp.float32(1.0)
    t2h, t2e = _two_prod(Sm1h, P1h)
    t2l = (Sm1h * P1l + Sl * P1h) + t2e
    t3h, t3l = _dadd(P2h, P2l, -t2h, -t2l)
    res = (t3h + t3l) / (Sh + Sl)
    out_ref[...] = res[0:1, 0:1]


_tc_tail = pl.pallas_call(
    _tail_body,
    out_shape=jax.ShapeDtypeStruct((1, 1), jnp.float32),
    in_specs=[
        pl.BlockSpec(memory_space=pltpu.VMEM),
        pl.BlockSpec(memory_space=pltpu.SMEM),
        pl.BlockSpec(memory_space=pltpu.SMEM),
    ],
    out_specs=pl.BlockSpec(memory_space=pltpu.VMEM),
    scratch_shapes=[
        pltpu.VMEM((8, 2 * NBP), jnp.float32),
        pltpu.VMEM((8, 2 * NBP), jnp.float32),
    ],
)


def _weights_bins(counts_raw, lo, hi):
    bw = (hi - lo) / jnp.float32(500.0)
    std = jnp.float32(1.0) * bw / 2.0
    denom = jnp.float32(math.sqrt(2.0 * math.pi)) * std
    q_hi = bw / denom
    p = q_hi * denom
    dh, dl = _split(q_hi)
    eh, el = _split(denom)
    perr = ((dh * eh - p) + dh * el + dl * eh) + dl * el
    q_lo = ((bw - p) - perr) / denom
    w = (counts_raw * q_hi + counts_raw * q_lo) / jnp.float32(NSAMP)
    centres = lo + bw * (jnp.arange(NB, dtype=jnp.float32) + 0.5)
    return w, centres


def _log_likelihood_ref(bins, cov):
    val = bins[:, None]
    loc = bins[None, :]
    lp = -0.5 * jnp.square((val - loc) / cov) - jnp.log(cov) - jnp.float32(HALF_LOG_2PI)
    return lp.sum(axis=0)


def kernel(XA_1d, XB_1d, likelihood_cov):
    x2 = jnp.concatenate([jnp.squeeze(XA_1d, axis=1), jnp.squeeze(XB_1d, axis=1)])
    counts, mm = _sc_hist(x2)
    weights_A, bins_A = _weights_bins(counts[:NB], mm[0], mm[1])
    weights_B, bins_B = _weights_bins(counts[NBP:NBP + NB], mm[16], mm[17])
    cov = likelihood_cov
    weights_AB = jnp.concatenate([weights_A, weights_B]) / (weights_A.sum() + weights_B.sum())
    bins_AB = jnp.concatenate([bins_A, bins_B])
    avg_log_llhd_A = (_log_likelihood_ref(bins_A, cov) * weights_A).sum()
    avg_log_llhd_B = (_log_likelihood_ref(bins_B, cov) * weights_B).sum()
    avg_log_llhd_AB = (_log_likelihood_ref(bins_AB, cov) * weights_AB).sum()
    return avg_log_llhd_AB - avg_log_llhd_A - avg_log_llhd_B


# speedup vs baseline: 1.7792x; 1.0002x over previous
import functools
import math

import jax
import jax.numpy as jnp
from jax import lax
from jax.experimental import pallas as pl
from jax.experimental.pallas import tpu as pltpu
from jax.experimental.pallas import tpu_sc as plsc

NB = 500
NBP = 512
NSAMP = 131072
NTILES = 16
CHUNK = NSAMP // NTILES
VECS = CHUNK // 16
TAPS = 3
HALF_LOG_2PI = 0.5 * math.log(2.0 * math.pi)


def _sc_hist_body(x2_hbm, counts_out, mm_out,
                  chunk_v, hist_v, counts_v, vec16_v, mmall_v, red_v,
                  out32_v, sh_mm, sh_counts):
    c = lax.axis_index("c")
    s = lax.axis_index("s")
    lane = lax.iota(jnp.int32, 16)

    pltpu.sync_copy(x2_hbm.at[pl.ds(c * NSAMP + s * CHUNK, CHUNK)], chunk_v)

    def mm_step(i, carry):
        mn, mx = carry
        for k in range(4):
            v = chunk_v[pl.ds(i * 64 + k * 16, 16)]
            mn = jnp.minimum(mn, v)
            mx = jnp.maximum(mx, v)
        return mn, mx

    v0 = chunk_v[pl.ds(0, 16)]
    mn_v, mx_v = lax.fori_loop(0, VECS // 4, mm_step, (v0, v0))
    vec16_v[...] = mn_v
    pltpu.sync_copy(vec16_v, sh_mm.at[pl.ds(s * 32, 16)])
    vec16_v[...] = mx_v
    pltpu.sync_copy(vec16_v, sh_mm.at[pl.ds(s * 32 + 16, 16)])
    plsc.subcore_barrier()
    pltpu.sync_copy(sh_mm, mmall_v)
    mn16 = mmall_v[pl.ds(0, 16)]
    mx16 = mmall_v[pl.ds(16, 16)]
    for r in range(1, NTILES):
        mn16 = jnp.minimum(mn16, mmall_v[pl.ds(r * 32, 16)])
        mx16 = jnp.maximum(mx16, mmall_v[pl.ds(r * 32 + 16, 16)])

    def lanefold(v, op):
        for shift in (8, 4, 2, 1):
            vec16_v[...] = v
            idx = (lane + shift) & 15
            v = op(v, plsc.load_gather(vec16_v, [idx]))
        return v

    gmn_v = lanefold(mn16, jnp.minimum)
    gmx_v = lanefold(mx16, jnp.maximum)
    dr_v = gmx_v - gmn_v
    bw_v = dr_v / jnp.float32(500.0)
    inv_bw_v = jnp.float32(500.0) / dr_v
    std_v = bw_v * jnp.float32(0.5)
    d2_v = (std_v * std_v) * jnp.float32(2.0)
    r_hi = jnp.float32(1.0) / d2_v
    p = r_hi * d2_v
    ah = r_hi * jnp.float32(4097.0)
    ah = ah - (ah - r_hi)
    al = r_hi - ah
    bh = d2_v * jnp.float32(4097.0)
    bh = bh - (bh - d2_v)
    bl = d2_v - bh
    perr = ((ah * bh - p) + ah * bl + al * bh) + al * bl
    r_lo = ((jnp.float32(1.0) - p) - perr) * r_hi
    nr_hi = -r_hi
    nr_lo = -r_lo


    zero16 = jnp.zeros((16,), jnp.float32)

    def zero_step(i, carry):
        hist_v[pl.ds(i * 16, 16)] = zero16
        return carry

    lax.fori_loop(0, (16 * NBP) // 16, zero_step, 0)

    lane_base = lane * NBP

    def hist_step(i, carry):
        for k in range(2):
            x = chunk_v[pl.ds(i * 32 + k * 16, 16)]
            u = (x - gmn_v) * inv_bw_v
            base = u.astype(jnp.int32)
            for d in range(-TAPS, TAPS + 1):
                ji = base + d
                jc0 = jnp.clip(ji, 0, NBP - 1)
                cj = gmn_v + bw_v * (ji.astype(jnp.float32) + jnp.float32(0.5))
                diff = x - cj
                sq = diff * diff
                q = sq * nr_hi + sq * nr_lo
                val = jnp.exp(q)
                m = (ji >= 0) & (ji < NB)
                plsc.addupdate_scatter(hist_v, [jc0 + lane_base], val, mask=m)
        return carry

    lax.fori_loop(0, VECS // 2, hist_step, 0)

    def lred_step(h, carry):
        a = hist_v[pl.ds(h * 16, 16)]
        for r in range(1, 16):
            a = a + hist_v[pl.ds(r * NBP + h * 16, 16)]
        counts_v[pl.ds(h * 16, 16)] = a
        return carry

    lax.fori_loop(0, NBP // 16, lred_step, 0)

    pltpu.sync_copy(counts_v, sh_counts.at[pl.ds(s * NBP, NBP)])
    plsc.subcore_barrier()
    for r in range(NTILES):
        pltpu.sync_copy(sh_counts.at[pl.ds(r * NBP + s * 32, 32)],
                        red_v.at[pl.ds(r * 32, 32)])
    for h in range(2):
        a = red_v[pl.ds(h * 16, 16)]
        for r in range(1, NTILES):
            a = a + red_v[pl.ds(r * 32 + h * 16, 16)]
        out32_v[pl.ds(h * 16, 16)] = a
    pltpu.sync_copy(out32_v, counts_out.at[pl.ds(c * NBP + s * 32, 32)])

    @pl.when(s == 0)
    def _():
        vec16_v[...] = jnp.where(lane == 0, gmn_v,
                                 jnp.where(lane == 1, gmx_v, jnp.float32(0.0)))
        pltpu.sync_copy(vec16_v, mm_out.at[pl.ds(c * 16, 16)])


_sc_hist = pl.kernel(
    _sc_hist_body,
    out_type=[
        jax.ShapeDtypeStruct((2 * NBP,), jnp.float32),
        jax.ShapeDtypeStruct((32,), jnp.float32),
    ],
    mesh=plsc.VectorSubcoreMesh(core_axis_name="c", subcore_axis_name="s"),
    compiler_params=pltpu.CompilerParams(needs_layout_passes=False),
    scratch_types=[
        pltpu.VMEM((CHUNK,), jnp.float32),
        pltpu.VMEM((16 * NBP,), jnp.float32),
        pltpu.VMEM((NBP,), jnp.float32),
        pltpu.VMEM((16,), jnp.float32),
        pltpu.VMEM((NTILES * 32,), jnp.float32),
        pltpu.VMEM((NTILES * 32,), jnp.float32),
        pltpu.VMEM((32,), jnp.float32),
        pltpu.VMEM_SHARED((NTILES * 32,), jnp.float32),
        pltpu.VMEM_SHARED((NTILES * NBP,), jnp.float32),
    ],
)


def _split(a):
    c = a * jnp.float32(4097.0)
    t = c - a
    hi = c - t
    return hi, a - hi


def _weights_bins(counts_raw, lo, hi):
    bw = (hi - lo) / jnp.float32(500.0)
    std = jnp.float32(1.0) * bw / 2.0
    denom = jnp.float32(math.sqrt(2.0 * math.pi)) * std
    q_hi = bw / denom
    p = q_hi * denom
    dh, dl = _split(q_hi)
    eh, el = _split(denom)
    perr = ((dh * eh - p) + dh * el + dl * eh) + dl * el
    q_lo = ((bw - p) - perr) / denom
    w = (counts_raw * q_hi + counts_raw * q_lo) / jnp.float32(NSAMP)
    centres = lo + bw * (jnp.arange(NB, dtype=jnp.float32) + 0.5)
    return w, centres


def _log_likelihood_ref(bins, cov):
    val = bins[:, None]
    loc = bins[None, :]
    lp = -0.5 * jnp.square((val - loc) / cov) - jnp.log(cov) - jnp.float32(HALF_LOG_2PI)
    return lp.sum(axis=0)


def kernel(XA_1d, XB_1d, likelihood_cov):
    x2 = jnp.concatenate([jnp.squeeze(XA_1d, axis=1), jnp.squeeze(XB_1d, axis=1)])
    counts, mm = _sc_hist(x2)
    weights_A, bins_A = _weights_bins(counts[:NB], mm[0], mm[1])
    weights_B, bins_B = _weights_bins(counts[NBP:NBP + NB], mm[16], mm[17])
    cov = likelihood_cov
    weights_AB = jnp.concatenate([weights_A, weights_B]) / (weights_A.sum() + weights_B.sum())
    bins_AB = jnp.concatenate([bins_A, bins_B])
    avg_log_llhd_A = (_log_likelihood_ref(bins_A, cov) * weights_A).sum()
    avg_log_llhd_B = (_log_likelihood_ref(bins_B, cov) * weights_B).sum()
    avg_log_llhd_AB = (_log_likelihood_ref(bins_AB, cov) * weights_AB).sum()
    return avg_log_llhd_AB - avg_log_llhd_A - avg_log_llhd_B
